# baseline (speedup 1.0000x reference)
import functools

import jax
import jax.numpy as jnp
from jax import lax
from jax.experimental import pallas as pl
from jax.experimental.pallas import tpu as pltpu
from jax.experimental.pallas import tpu_sc as plsc

EMBED = 64
N_BINS = 256
_info = plsc.get_sparse_core_info()
NC, NS = _info.num_cores, _info.num_subcores


def _make_sc_lookup(B: int, C: int):
    b_per_sc = B // NC
    win_rows = NS * C
    windows = b_per_sc // win_rows
    assert windows * win_rows * NC == B
    mesh = plsc.VectorSubcoreMesh(core_axis_name="c", subcore_axis_name="s")

    @functools.partial(
        pl.kernel,
        out_type=jax.ShapeDtypeStruct((B * EMBED,), jnp.float32),
        mesh=mesh,
        scratch_types=[
            pltpu.VMEM((N_BINS * EMBED,), jnp.float32),
            pltpu.VMEM((2, C), jnp.int32),
            pltpu.VMEM((C * EMBED,), jnp.float32),
            pltpu.VMEM_SHARED((2, win_rows * EMBED), jnp.float32),
            pltpu.SemaphoreType.DMA((2,)),
            pltpu.SemaphoreType.DMA((2,)),
        ],
        compiler_params=pltpu.CompilerParams(use_tc_tiling_on_sc=False,
                                             needs_layout_passes=False),
    )
    def sc_lookup(x_hbm, table_hbm, out_hbm, table_v, idx_v, rows_v, stage_s,
                  sem_idx, sem_big):
        sc = lax.axis_index("c")
        tid = lax.axis_index("s")
        base0 = sc * b_per_sc + tid * C

        pltpu.sync_copy(table_hbm, table_v)

        pltpu.async_copy(x_hbm.at[pl.ds(base0, C)], idx_v.at[0],
                         sem_idx.at[0])

        @pl.loop(0, windows)
        def _window(w):
            b = w % 2
            nb = 1 - b

            @pl.when(w + 1 < windows)
            def _prefetch():
                nbase = base0 + (w + 1) * win_rows
                pltpu.async_copy(x_hbm.at[pl.ds(nbase, C)], idx_v.at[nb],
                                 sem_idx.at[nb])

            pltpu.make_async_copy(x_hbm.at[pl.ds(base0, C)], idx_v.at[b],
                                  sem_idx.at[b]).wait()

            @plsc.parallel_loop(0, C // 16)
            def _i(i):
                idx16 = idx_v[b, pl.ds(i * 16, 16)] * EMBED
                for l in range(16):
                    src = idx16[l]
                    dst = (i * 16 + l) * EMBED
                    for k in range(EMBED // 16):
                        rows_v[pl.ds(dst + k * 16, 16)] = (
                            table_v[pl.ds(src + k * 16, 16)])

            @pl.when((tid == 0) & (w >= 2))
            def _drain():
                obase = (sc * b_per_sc + (w - 2) * win_rows) * EMBED
                pltpu.make_async_copy(
                    stage_s.at[b], out_hbm.at[pl.ds(obase, win_rows * EMBED)],
                    sem_big.at[b]).wait()

            plsc.subcore_barrier()

            pltpu.sync_copy(rows_v,
                            stage_s.at[b, pl.ds(tid * C * EMBED, C * EMBED)])

            plsc.subcore_barrier()

            @pl.when(tid == 0)
            def _fire():
                obase = (sc * b_per_sc + w * win_rows) * EMBED
                pltpu.async_copy(
                    stage_s.at[b], out_hbm.at[pl.ds(obase, win_rows * EMBED)],
                    sem_big.at[b])

        @pl.when(tid == 0)
        def _tail():
            @pl.loop(windows - 2, windows)
            def _t(w):
                b = w % 2
                obase = (sc * b_per_sc + w * win_rows) * EMBED
                pltpu.make_async_copy(
                    stage_s.at[b], out_hbm.at[pl.ds(obase, win_rows * EMBED)],
                    sem_big.at[b]).wait()

    return sc_lookup


def kernel(x, table):
    lead = x.shape[:-1]
    k = x.shape[-1]
    B = 1
    for s in x.shape:
        B *= s
    xf = x.reshape(B).astype(jnp.int32)
    out = _make_sc_lookup(B, 512)(xf, table.reshape(-1))
    return out.reshape(lead + (k * EMBED,))
